# baseline (speedup 1.0000x reference)
import functools

import jax
import jax.numpy as jnp
import numpy as np
from jax import lax
from jax.experimental import pallas as pl
from jax.experimental.pallas import tpu as pltpu
from jax.experimental.pallas import tpu_sc as plsc

_RATE = 0.1

_drop_cache = {}


def _drop_vec(b):
    if b not in _drop_cache:
        keep = 1.0 - _RATE
        with jax.ensure_compile_time_eval():
            u = jax.random.uniform(jax.random.key(1), (b, 1), dtype=jnp.float32)
            drop = jnp.ones((b, 1), jnp.float32) / keep * jnp.floor(keep + u)
        _drop_cache[b] = np.asarray(drop)[:, 0]
    return _drop_cache[b]


def _sc_full_body(nn_hbm, x_hbm, o_hbm, nn_v, xbuf, obuf, *, dd, d,
                  rows_pc, chunk):
    pltpu.sync_copy(nn_hbm, nn_v)
    nn = nn_v[...]
    steps = []
    acc = nn[0]
    for k in range(1, len(dd)):
        if dd[k] != 0.0:
            steps.append((acc, dd[k]))
        if k < len(dd) - 1:
            acc = acc + nn[k]
    wid = lax.axis_index("s") * 2 + lax.axis_index("c")
    base = wid * rows_pc
    for c in range(rows_pc // chunk):
        r0 = base + c * chunk
        pltpu.sync_copy(x_hbm.at[pl.ds(r0, chunk)], xbuf)

        def row_body(r, carry):
            row = r0 + r
            s = jnp.float32(dd[0])
            for e_k, dd_k in steps:
                s = s + jnp.where(row >= e_k, jnp.float32(dd_k),
                                  jnp.float32(0.0))
            for c16 in range(d // 16):
                v = xbuf[r, pl.ds(c16 * 16, 16)]
                obuf[r, pl.ds(c16 * 16, 16)] = v * s
            return carry

        lax.fori_loop(0, chunk, row_body, 0)
        pltpu.sync_copy(obuf, o_hbm.at[pl.ds(r0, chunk)])


def kernel(inputs, n_node):
    n, d = inputs.shape
    b = n_node.shape[0]
    drop = _drop_vec(b)
    dd = [float(drop[0])] + [float(drop[k] - drop[k - 1]) for k in range(1, b)]

    nw = 32
    rows_pc = n // nw
    chunk = 128
    mesh = plsc.VectorSubcoreMesh(core_axis_name="c", subcore_axis_name="s")
    return pl.kernel(
        functools.partial(_sc_full_body, dd=tuple(dd), d=d,
                          rows_pc=rows_pc, chunk=chunk),
        out_type=jax.ShapeDtypeStruct((n, d), jnp.float32),
        mesh=mesh,
        scratch_types=[
            pltpu.VMEM((b,), jnp.int32),
            pltpu.VMEM((chunk, d), jnp.float32),
            pltpu.VMEM((chunk, d), jnp.float32),
        ],
    )(n_node.astype(jnp.int32), inputs)


# speedup vs baseline: 1.2412x; 1.2412x over previous
import functools

import jax
import jax.numpy as jnp
import numpy as np
from jax import lax
from jax.experimental import pallas as pl
from jax.experimental.pallas import tpu as pltpu
from jax.experimental.pallas import tpu_sc as plsc

_RATE = 0.1

_drop_cache = {}


def _drop_vec(b):
    if b not in _drop_cache:
        keep = 1.0 - _RATE
        with jax.ensure_compile_time_eval():
            u = jax.random.uniform(jax.random.key(1), (b, 1), dtype=jnp.float32)
            drop = jnp.ones((b, 1), jnp.float32) / keep * jnp.floor(keep + u)
        _drop_cache[b] = np.asarray(drop)[:, 0]
    return _drop_cache[b]


def _sc_full_body(nn_hbm, x_hbm, o_hbm, nn_v, xb0, xb1, ob0, ob1,
                  si0, si1, so0, so1, *, dd, d, rows_pc, chunk):
    pltpu.sync_copy(nn_hbm, nn_v)
    nn = nn_v[...]
    steps = []
    acc = nn[0]
    for k in range(1, len(dd)):
        if dd[k] != 0.0:
            steps.append((acc, dd[k]))
        if k < len(dd) - 1:
            acc = acc + nn[k]
    wid = lax.axis_index("s") * 2 + lax.axis_index("c")
    base = wid * rows_pc
    xb, ob, si, so = (xb0, xb1), (ob0, ob1), (si0, si1), (so0, so1)
    nchunk = rows_pc // chunk
    in_flight = {}
    out_flight = {}
    in_flight[0] = pltpu.async_copy(
        x_hbm.at[pl.ds(base, chunk)], xb[0], si[0])
    for c in range(nchunk):
        b = c % 2
        if c + 1 < nchunk:
            in_flight[c + 1] = pltpu.async_copy(
                x_hbm.at[pl.ds(base + (c + 1) * chunk, chunk)],
                xb[1 - b], si[1 - b])
        in_flight.pop(c).wait()
        if c >= 2:
            out_flight.pop(c - 2).wait()
        r0 = base + c * chunk

        def row_body(r, carry):
            row = r0 + r
            s = jnp.float32(dd[0])
            for e_k, dd_k in steps:
                s = s + jnp.where(row >= e_k, jnp.float32(dd_k),
                                  jnp.float32(0.0))
            for c16 in range(d // 16):
                v = xb[b][r, pl.ds(c16 * 16, 16)]
                ob[b][r, pl.ds(c16 * 16, 16)] = v * s
            return carry

        lax.fori_loop(0, chunk, row_body, 0)
        out_flight[c] = pltpu.async_copy(
            ob[b], o_hbm.at[pl.ds(r0, chunk)], so[b])
    for c in sorted(out_flight):
        out_flight.pop(c).wait()


def kernel(inputs, n_node):
    n, d = inputs.shape
    b = n_node.shape[0]
    drop = _drop_vec(b)
    dd = [float(drop[0])] + [float(drop[k] - drop[k - 1]) for k in range(1, b)]

    nw = 32
    rows_pc = n // nw
    chunk = 64
    mesh = plsc.VectorSubcoreMesh(core_axis_name="c", subcore_axis_name="s")
    return pl.kernel(
        functools.partial(_sc_full_body, dd=tuple(dd), d=d,
                          rows_pc=rows_pc, chunk=chunk),
        out_type=jax.ShapeDtypeStruct((n, d), jnp.float32),
        mesh=mesh,
        scratch_types=[
            pltpu.VMEM((b,), jnp.int32),
            pltpu.VMEM((chunk, d), jnp.float32),
            pltpu.VMEM((chunk, d), jnp.float32),
            pltpu.VMEM((chunk, d), jnp.float32),
            pltpu.VMEM((chunk, d), jnp.float32),
            pltpu.SemaphoreType.DMA,
            pltpu.SemaphoreType.DMA,
            pltpu.SemaphoreType.DMA,
            pltpu.SemaphoreType.DMA,
        ],
    )(n_node.astype(jnp.int32), inputs)


# speedup vs baseline: 2.7757x; 2.2362x over previous
import functools

import jax
import jax.numpy as jnp
import numpy as np
from jax.experimental import pallas as pl
from jax.experimental.pallas import tpu as pltpu

_RATE = 0.1


def _body(nn_ref, x_ref, o_ref, *, rows_per_blk, dd):
    s8 = rows_per_blk // 128
    row0 = pl.program_id(0) * rows_per_blk
    rows = (jax.lax.broadcasted_iota(jnp.int32, (s8, 128), 0) * 128
            + jax.lax.broadcasted_iota(jnp.int32, (s8, 128), 1) + row0)
    scale = jnp.full((s8, 128), dd[0], dtype=jnp.float32)
    e_k = nn_ref[0]
    for k in range(1, len(dd)):
        if dd[k] != 0.0:
            scale = scale + jnp.where(rows >= e_k, jnp.float32(dd[k]), 0.0)
        e_k = e_k + nn_ref[k]
    o_ref[...] = x_ref[...] * scale[:, :, None]


_drop_cache = {}


def _drop_vec(b):
    if b not in _drop_cache:
        keep = 1.0 - _RATE
        with jax.ensure_compile_time_eval():
            u = jax.random.uniform(jax.random.key(1), (b, 1), dtype=jnp.float32)
            drop = jnp.ones((b, 1), jnp.float32) / keep * jnp.floor(keep + u)
        _drop_cache[b] = np.asarray(drop)[:, 0]
    return _drop_cache[b]


def kernel(inputs, n_node):
    n, d = inputs.shape
    b = n_node.shape[0]
    drop = _drop_vec(b)
    dd = [float(drop[0])] + [float(drop[k] - drop[k - 1]) for k in range(1, b)]

    rows_per_blk = 8192
    grid = n // rows_per_blk
    x3 = inputs.reshape(n // 128, 128, d)
    out = pl.pallas_call(
        functools.partial(_body, rows_per_blk=rows_per_blk, dd=dd),
        grid=(grid,),
        in_specs=[
            pl.BlockSpec(memory_space=pltpu.SMEM),
            pl.BlockSpec((rows_per_blk // 128, 128, d), lambda i: (i, 0, 0)),
        ],
        out_specs=pl.BlockSpec((rows_per_blk // 128, 128, d), lambda i: (i, 0, 0)),
        out_shape=jax.ShapeDtypeStruct((n // 128, 128, d), inputs.dtype),
    )(n_node.astype(jnp.int32), x3)
    return out.reshape(n, d)
